# baseline (speedup 1.0000x reference)
import functools

import jax
import jax.numpy as jnp
from jax import lax
from jax.experimental import pallas as pl
from jax.experimental.pallas import tpu as pltpu
from jax.experimental.pallas import tpu_sc as plsc

_N = 10000
_C = 128
_E = 320000
_EPS = 1e-5

_NC = 2
_NS = 16
_NW = _NC * _NS
_K = 256
_CPW = 40
_EPAD = _NW * _CPW * _K
_NP = _N + 8
_RPT = 624
_TAIL = _N - _RPT * _NS

_mesh = plsc.VectorSubcoreMesh(core_axis_name="c", subcore_axis_name="s")


@functools.partial(
    pl.kernel,
    out_type=jax.ShapeDtypeStruct((_NC, _N, _C), jnp.float32),
    mesh=_mesh,
    scratch_types=[
        pltpu.VMEM((_K,), jnp.int32),
        pltpu.VMEM((_K,), jnp.int32),
        pltpu.VMEM((_K, _C), jnp.float32),
        pltpu.VMEM_SHARED((_NP, _C), jnp.float32),
        pltpu.SemaphoreType.DMA,
    ],
)
def _sc_segment_sum(srce_ref, dste_ref, wedge_ref, zeros_ref, out_ref,
                    si, di, rows, acc_sh, g0):
    cid = lax.axis_index("c")
    sid = lax.axis_index("s")
    wid = cid * _NS + sid
    e0 = wid * _CPW * _K

    r0 = sid * _RPT
    pltpu.sync_copy(zeros_ref.at[pl.ds(r0, _RPT)], acc_sh.at[pl.ds(r0, _RPT)])

    @pl.when(sid == _NS - 1)
    def _():
        t0 = _RPT * _NS
        pltpu.sync_copy(zeros_ref.at[pl.ds(t0, _NP - t0)],
                        acc_sh.at[pl.ds(t0, _NP - t0)])

    plsc.subcore_barrier()

    def body(c, carry):
        base = e0 + c * _K
        pltpu.sync_copy(srce_ref.at[pl.ds(base, _K)], si)
        pltpu.sync_copy(dste_ref.at[pl.ds(base, _K)], di)
        pltpu.async_copy(wedge_ref.at[si], rows, g0).wait()
        pltpu.sync_copy(rows, acc_sh.at[di], add=True)
        return carry

    lax.fori_loop(0, _CPW, body, 0)

    plsc.subcore_barrier()
    pltpu.sync_copy(acc_sh.at[pl.ds(r0, _RPT)],
                    out_ref.at[cid, pl.ds(r0, _RPT)])

    @pl.when(sid == _NS - 1)
    def _():
        t0 = _RPT * _NS
        pltpu.sync_copy(acc_sh.at[pl.ds(t0, _TAIL)],
                        out_ref.at[cid, pl.ds(t0, _TAIL)])


def _mm(a, b):
    return lax.dot_general(a, b, (((1,), (1,)), ((), ())),
                           preferred_element_type=jnp.float32)


def _style(h_in, sty, lW, lb, aWg, aWb, abg, abb, nzs):
    h = _mm(h_in, lW) + lb + nzs
    gamma = _mm(sty, aWg) + abg
    beta = _mm(sty, aWb) + abb
    mu = jnp.mean(h, axis=0, keepdims=True)
    var = jnp.mean((h - mu) * (h - mu), axis=0, keepdims=True)
    hn = (h - mu) * lax.rsqrt(var + _EPS)
    h = gamma * hn + beta
    return jnp.where(h >= 0, h, 0.01 * h)


def _tc_body(acc_ref, x_ref, style_ref, bedge_ref, Wc1_ref, bc1_ref,
             Wc2_ref, bc2_ref,
             l0W_ref, l0b_ref, a0Wg_ref, a0Wb_ref, a0bg_ref, a0bb_ref, nzs0_ref,
             l1W_ref, l1b_ref, a1Wg_ref, a1Wb_ref, a1bg_ref, a1bb_ref, nzs1_ref,
             l2W_ref, l2b_ref, a2Wg_ref, a2Wb_ref, a2bg_ref, a2bb_ref, nzs2_ref,
             out_ref):
    x = x_ref[...]
    sty = style_ref[...]
    out = acc_ref[0] + acc_ref[1] + bedge_ref[...]
    out = out + _mm(out, Wc1_ref[...]) + bc1_ref[...]
    xm = _style(x, sty, l0W_ref[...], l0b_ref[...], a0Wg_ref[...],
                a0Wb_ref[...], a0bg_ref[...], a0bb_ref[...], nzs0_ref[...])
    out = out + xm
    out = out + _mm(xm, Wc2_ref[...]) + bc2_ref[...]
    out = jnp.maximum(out, 0.0)
    out = _style(out, sty, l1W_ref[...], l1b_ref[...], a1Wg_ref[...],
                 a1Wb_ref[...], a1bg_ref[...], a1bb_ref[...], nzs1_ref[...])
    out = _style(out, sty, l2W_ref[...], l2b_ref[...], a2Wg_ref[...],
                 a2Wb_ref[...], a2bg_ref[...], a2bb_ref[...], nzs2_ref[...])
    out_ref[...] = out


_tc_call = pl.pallas_call(
    _tc_body,
    out_shape=jax.ShapeDtypeStruct((_N, _C), jnp.float32),
)


def kernel(x, edge_index, style, W_edge, b_edge, Wc1, bc1, Wc2, bc2,
           l0W, l0b, a0W, a0b, ns0, nz0,
           l1W, l1b, a1W, a1b, ns1, nz1,
           l2W, l2b, a2W, a2b, ns2, nz2):
    ei = edge_index.astype(jnp.int32)
    npad = _EPAD - _E
    srcp = jnp.concatenate([ei[0], jnp.zeros((npad,), jnp.int32)])
    dstp = jnp.concatenate([ei[1], jnp.full((npad,), _N, jnp.int32)])
    zeros = jnp.zeros((_NP, _C), jnp.float32)
    acc = _sc_segment_sum(srcp, dstp, W_edge, zeros)

    def prep(aW, ab, ns, nz):
        return (aW[:_C], aW[_C:], ab[:_C].reshape(1, _C),
                ab[_C:].reshape(1, _C), (ns * nz).reshape(1, _C))

    a0Wg, a0Wb, a0bg, a0bb, nzs0 = prep(a0W, a0b, ns0, nz0)
    a1Wg, a1Wb, a1bg, a1bb, nzs1 = prep(a1W, a1b, ns1, nz1)
    a2Wg, a2Wb, a2bg, a2bb, nzs2 = prep(a2W, a2b, ns2, nz2)

    return _tc_call(
        acc, x, style, b_edge.reshape(1, _C), Wc1, bc1.reshape(1, _C),
        Wc2, bc2.reshape(1, _C),
        l0W, l0b.reshape(1, _C), a0Wg, a0Wb, a0bg, a0bb, nzs0,
        l1W, l1b.reshape(1, _C), a1Wg, a1Wb, a1bg, a1bb, nzs1,
        l2W, l2b.reshape(1, _C), a2Wg, a2Wb, a2bg, a2bb, nzs2)


# speedup vs baseline: 3.0997x; 3.0997x over previous
import functools

import jax
import jax.numpy as jnp
from jax import lax
from jax.experimental import pallas as pl
from jax.experimental.pallas import tpu as pltpu
from jax.experimental.pallas import tpu_sc as plsc

_N = 10000
_C = 128
_E = 320000
_EPS = 1e-5

_NC = 2
_NS = 16
_NW = _NC * _NS
_K = 128
_EPW = _E // _NW
_FULL = 78
_REM = _EPW - _FULL * _K
_EPAD = _E + 2 * _K
_NP = _N
_RPT = 624
_TAIL = _N - _RPT * _NS

_mesh = plsc.VectorSubcoreMesh(core_axis_name="c", subcore_axis_name="s")


@functools.partial(
    pl.kernel,
    out_type=jax.ShapeDtypeStruct((_NC, _N, _C), jnp.float32),
    mesh=_mesh,
    scratch_types=[
        pltpu.VMEM((_K,), jnp.int32),
        pltpu.VMEM((_K,), jnp.int32),
        pltpu.VMEM((_K,), jnp.int32),
        pltpu.VMEM((_K,), jnp.int32),
        pltpu.VMEM((_K, _C), jnp.float32),
        pltpu.VMEM((_K, _C), jnp.float32),
        pltpu.VMEM((_REM,), jnp.int32),
        pltpu.VMEM((_REM,), jnp.int32),
        pltpu.VMEM((_REM, _C), jnp.float32),
        pltpu.VMEM_SHARED((_NP, _C), jnp.float32),
        pltpu.SemaphoreType.DMA,
        pltpu.SemaphoreType.DMA,
        pltpu.SemaphoreType.DMA,
        pltpu.SemaphoreType.DMA,
    ],
)
def _sc_segment_sum(srce_ref, dste_ref, wedge_ref, zeros_ref, out_ref,
                    si0, si1, di0, di1, rows0, rows1, sit, dit, rowst,
                    acc_sh, gi0, gi1, g0, g1):
    cid = lax.axis_index("c")
    sid = lax.axis_index("s")
    wid = cid * _NS + sid
    e0 = wid * _EPW

    def idx_start(c, si, di, sem):
        base = e0 + c * _K
        pltpu.async_copy(srce_ref.at[pl.ds(base, _K)], si, sem)
        pltpu.async_copy(dste_ref.at[pl.ds(base, _K)], di, sem)

    def idx_wait(si, di, sem):
        pltpu.make_async_copy(srce_ref.at[pl.ds(0, _K)], si, sem).wait()
        pltpu.make_async_copy(dste_ref.at[pl.ds(0, _K)], di, sem).wait()

    idx_start(0, si0, di0, gi0)
    idx_start(1, si1, di1, gi1)

    r0 = sid * _RPT
    pltpu.sync_copy(zeros_ref.at[pl.ds(r0, _RPT)], acc_sh.at[pl.ds(r0, _RPT)])

    @pl.when(sid == _NS - 1)
    def _():
        t0 = _RPT * _NS
        pltpu.sync_copy(zeros_ref.at[pl.ds(t0, _NP - t0)],
                        acc_sh.at[pl.ds(t0, _NP - t0)])

    idx_wait(si0, di0, gi0)
    pltpu.async_copy(wedge_ref.at[si0], rows0, g0)
    plsc.subcore_barrier()

    def body(i, carry):
        c = 2 * i
        pltpu.make_async_copy(wedge_ref.at[si0], rows0, g0).wait()
        idx_wait(si1, di1, gi1)
        pltpu.async_copy(wedge_ref.at[si1], rows1, g1)
        pltpu.sync_copy(rows0, acc_sh.at[di0], add=True)
        idx_start(c + 2, si0, di0, gi0)
        pltpu.make_async_copy(wedge_ref.at[si1], rows1, g1).wait()
        idx_wait(si0, di0, gi0)
        pltpu.async_copy(wedge_ref.at[si0], rows0, g0)
        pltpu.sync_copy(rows1, acc_sh.at[di1], add=True)
        idx_start(c + 3, si1, di1, gi1)
        return carry

    lax.fori_loop(0, _FULL // 2, body, 0)
    pltpu.make_async_copy(wedge_ref.at[si0], rows0, g0).wait()
    idx_wait(si1, di1, gi1)

    tbase = e0 + _FULL * _K
    pltpu.sync_copy(srce_ref.at[pl.ds(tbase, _REM)], sit)
    pltpu.sync_copy(dste_ref.at[pl.ds(tbase, _REM)], dit)
    pltpu.async_copy(wedge_ref.at[sit], rowst, g0).wait()
    pltpu.sync_copy(rowst, acc_sh.at[dit], add=True)

    plsc.subcore_barrier()
    pltpu.sync_copy(acc_sh.at[pl.ds(r0, _RPT)],
                    out_ref.at[cid, pl.ds(r0, _RPT)])

    @pl.when(sid == _NS - 1)
    def _():
        t0 = _RPT * _NS
        pltpu.sync_copy(acc_sh.at[pl.ds(t0, _TAIL)],
                        out_ref.at[cid, pl.ds(t0, _TAIL)])


def _mm(a, b):
    return lax.dot_general(a, b, (((1,), (1,)), ((), ())),
                           preferred_element_type=jnp.float32)


def _style(h_in, sty, lW, lb, aWg, aWb, abg, abb, nzs):
    h = _mm(h_in, lW) + lb + nzs
    gamma = _mm(sty, aWg) + abg
    beta = _mm(sty, aWb) + abb
    mu = jnp.mean(h, axis=0, keepdims=True)
    var = jnp.mean((h - mu) * (h - mu), axis=0, keepdims=True)
    hn = (h - mu) * lax.rsqrt(var + _EPS)
    h = gamma * hn + beta
    return jnp.where(h >= 0, h, 0.01 * h)


def _tc_body(acc_ref, x_ref, style_ref, bedge_ref, Wc1_ref, bc1_ref,
             Wc2_ref, bc2_ref,
             l0W_ref, l0b_ref, a0Wg_ref, a0Wb_ref, a0bg_ref, a0bb_ref, nzs0_ref,
             l1W_ref, l1b_ref, a1Wg_ref, a1Wb_ref, a1bg_ref, a1bb_ref, nzs1_ref,
             l2W_ref, l2b_ref, a2Wg_ref, a2Wb_ref, a2bg_ref, a2bb_ref, nzs2_ref,
             out_ref):
    x = x_ref[...]
    sty = style_ref[...]
    out = acc_ref[0] + acc_ref[1] + bedge_ref[...]
    out = out + _mm(out, Wc1_ref[...]) + bc1_ref[...]
    xm = _style(x, sty, l0W_ref[...], l0b_ref[...], a0Wg_ref[...],
                a0Wb_ref[...], a0bg_ref[...], a0bb_ref[...], nzs0_ref[...])
    out = out + xm
    out = out + _mm(xm, Wc2_ref[...]) + bc2_ref[...]
    out = jnp.maximum(out, 0.0)
    out = _style(out, sty, l1W_ref[...], l1b_ref[...], a1Wg_ref[...],
                 a1Wb_ref[...], a1bg_ref[...], a1bb_ref[...], nzs1_ref[...])
    out = _style(out, sty, l2W_ref[...], l2b_ref[...], a2Wg_ref[...],
                 a2Wb_ref[...], a2bg_ref[...], a2bb_ref[...], nzs2_ref[...])
    out_ref[...] = out


_tc_call = pl.pallas_call(
    _tc_body,
    out_shape=jax.ShapeDtypeStruct((_N, _C), jnp.float32),
)


def kernel(x, edge_index, style, W_edge, b_edge, Wc1, bc1, Wc2, bc2,
           l0W, l0b, a0W, a0b, ns0, nz0,
           l1W, l1b, a1W, a1b, ns1, nz1,
           l2W, l2b, a2W, a2b, ns2, nz2):
    ei = edge_index.astype(jnp.int32)
    npad = _EPAD - _E
    srcp = jnp.concatenate([ei[0], jnp.zeros((npad,), jnp.int32)])
    dstp = jnp.concatenate([ei[1], jnp.zeros((npad,), jnp.int32)])
    zeros = jnp.zeros((_NP, _C), jnp.float32)
    acc = _sc_segment_sum(srcp, dstp, W_edge, zeros)

    def prep(aW, ab, ns, nz):
        return (aW[:_C], aW[_C:], ab[:_C].reshape(1, _C),
                ab[_C:].reshape(1, _C), (ns * nz).reshape(1, _C))

    a0Wg, a0Wb, a0bg, a0bb, nzs0 = prep(a0W, a0b, ns0, nz0)
    a1Wg, a1Wb, a1bg, a1bb, nzs1 = prep(a1W, a1b, ns1, nz1)
    a2Wg, a2Wb, a2bg, a2bb, nzs2 = prep(a2W, a2b, ns2, nz2)

    return _tc_call(
        acc, x, style, b_edge.reshape(1, _C), Wc1, bc1.reshape(1, _C),
        Wc2, bc2.reshape(1, _C),
        l0W, l0b.reshape(1, _C), a0Wg, a0Wb, a0bg, a0bb, nzs0,
        l1W, l1b.reshape(1, _C), a1Wg, a1Wb, a1bg, a1bb, nzs1,
        l2W, l2b.reshape(1, _C), a2Wg, a2Wb, a2bg, a2bb, nzs2)


# speedup vs baseline: 3.1507x; 1.0165x over previous
import functools

import jax
import jax.numpy as jnp
from jax import lax
from jax.experimental import pallas as pl
from jax.experimental.pallas import tpu as pltpu
from jax.experimental.pallas import tpu_sc as plsc

_N = 10000
_C = 128
_E = 320000
_EPS = 1e-5

_NC = 2
_NS = 16
_NW = _NC * _NS
_K = 128
_EPW = _E // _NW
_FULL = 78
_REM = _EPW - _FULL * _K
_EPAD = _E + 2 * _K
_NP = _N
_RPT = 624
_TAIL = _N - _RPT * _NS

_mesh = plsc.VectorSubcoreMesh(core_axis_name="c", subcore_axis_name="s")


@functools.partial(
    pl.kernel,
    out_type=jax.ShapeDtypeStruct((_NC, _N, _C), jnp.float32),
    mesh=_mesh,
    scratch_types=[
        pltpu.VMEM((_K,), jnp.int32),
        pltpu.VMEM((_K,), jnp.int32),
        pltpu.VMEM((_K,), jnp.int32),
        pltpu.VMEM((_K,), jnp.int32),
        pltpu.VMEM((_K, _C), jnp.float32),
        pltpu.VMEM((_K, _C), jnp.float32),
        pltpu.VMEM((_REM,), jnp.int32),
        pltpu.VMEM((_REM,), jnp.int32),
        pltpu.VMEM((_REM, _C), jnp.float32),
        pltpu.VMEM_SHARED((_NP, _C), jnp.float32),
        pltpu.SemaphoreType.DMA,
        pltpu.SemaphoreType.DMA,
        pltpu.SemaphoreType.DMA,
        pltpu.SemaphoreType.DMA,
    ],
)
def _sc_segment_sum(srce_ref, dste_ref, wedge_ref, zeros_ref, out_ref,
                    si0, si1, di0, di1, rows0, rows1, sit, dit, rowst,
                    acc_sh, gi0, gi1, g0, g1):
    cid = lax.axis_index("c")
    sid = lax.axis_index("s")
    wid = cid * _NS + sid
    e0 = wid * _EPW

    def idx_start(c, si, di, sem):
        base = e0 + c * _K
        pltpu.async_copy(srce_ref.at[pl.ds(base, _K)], si, sem)
        pltpu.async_copy(dste_ref.at[pl.ds(base, _K)], di, sem)

    def idx_wait(si, di, sem):
        pltpu.make_async_copy(srce_ref.at[pl.ds(0, _K)], si, sem).wait()
        pltpu.make_async_copy(dste_ref.at[pl.ds(0, _K)], di, sem).wait()

    idx_start(0, si0, di0, gi0)
    idx_start(1, si1, di1, gi1)

    r0 = sid * _RPT
    pltpu.sync_copy(zeros_ref.at[pl.ds(r0, _RPT)], acc_sh.at[pl.ds(r0, _RPT)])

    @pl.when(sid == _NS - 1)
    def _():
        t0 = _RPT * _NS
        pltpu.sync_copy(zeros_ref.at[pl.ds(t0, _NP - t0)],
                        acc_sh.at[pl.ds(t0, _NP - t0)])

    idx_wait(si0, di0, gi0)
    pltpu.async_copy(wedge_ref.at[si0], rows0, g0)
    plsc.subcore_barrier()

    def body(i, carry):
        c = 2 * i
        pltpu.make_async_copy(wedge_ref.at[si0], rows0, g0).wait()
        idx_wait(si1, di1, gi1)
        pltpu.async_copy(wedge_ref.at[si1], rows1, g1)
        pltpu.sync_copy(rows0, acc_sh.at[di0], add=True)
        idx_start(c + 2, si0, di0, gi0)
        pltpu.make_async_copy(wedge_ref.at[si1], rows1, g1).wait()
        idx_wait(si0, di0, gi0)
        pltpu.async_copy(wedge_ref.at[si0], rows0, g0)
        pltpu.sync_copy(rows1, acc_sh.at[di1], add=True)
        idx_start(c + 3, si1, di1, gi1)
        return carry

    lax.fori_loop(0, _FULL // 2, body, 0)
    pltpu.make_async_copy(wedge_ref.at[si0], rows0, g0).wait()
    idx_wait(si1, di1, gi1)

    tbase = e0 + _FULL * _K
    pltpu.sync_copy(srce_ref.at[pl.ds(tbase, _REM)], sit)
    pltpu.sync_copy(dste_ref.at[pl.ds(tbase, _REM)], dit)
    pltpu.async_copy(wedge_ref.at[sit], rowst, g0).wait()
    pltpu.sync_copy(rowst, acc_sh.at[dit], add=True)

    plsc.subcore_barrier()
    pltpu.sync_copy(acc_sh.at[pl.ds(r0, _RPT)],
                    out_ref.at[cid, pl.ds(r0, _RPT)])

    @pl.when(sid == _NS - 1)
    def _():
        t0 = _RPT * _NS
        pltpu.sync_copy(acc_sh.at[pl.ds(t0, _TAIL)],
                        out_ref.at[cid, pl.ds(t0, _TAIL)])


def _mm(a, b):
    return lax.dot_general(a, b, (((1,), (1,)), ((), ())),
                           preferred_element_type=jnp.float32)


def _style(h_in, sty, lW, lb, aWg, aWb, abg, abb, nzs):
    h = _mm(h_in, lW) + lb + nzs
    gamma = _mm(sty, aWg) + abg
    beta = _mm(sty, aWb) + abb
    mu = jnp.mean(h, axis=0, keepdims=True)
    var = jnp.mean((h - mu) * (h - mu), axis=0, keepdims=True)
    hn = (h - mu) * lax.rsqrt(var + _EPS)
    h = gamma * hn + beta
    return jnp.where(h >= 0, h, 0.01 * h)


def _tc1_body(x_ref, style_ref, l0W_ref, l0b_ref, a0Wg_ref, a0Wb_ref,
              a0bg_ref, a0bb_ref, nzs0_ref, xm_ref):
    xm_ref[...] = _style(x_ref[...], style_ref[...], l0W_ref[...],
                         l0b_ref[...], a0Wg_ref[...], a0Wb_ref[...],
                         a0bg_ref[...], a0bb_ref[...], nzs0_ref[...])


_tc1_call = pl.pallas_call(
    _tc1_body,
    out_shape=jax.ShapeDtypeStruct((_N, _C), jnp.float32),
)


def _tc2_body(acc_ref, xm_ref, style_ref, bedge_ref, Wc1_ref, bc1_ref,
              Wc2_ref, bc2_ref,
              l1W_ref, l1b_ref, a1Wg_ref, a1Wb_ref, a1bg_ref, a1bb_ref,
              nzs1_ref,
              l2W_ref, l2b_ref, a2Wg_ref, a2Wb_ref, a2bg_ref, a2bb_ref,
              nzs2_ref, out_ref):
    sty = style_ref[...]
    xm = xm_ref[...]
    out = acc_ref[0] + acc_ref[1] + bedge_ref[...]
    out = out + _mm(out, Wc1_ref[...]) + bc1_ref[...]
    out = out + xm
    out = out + _mm(xm, Wc2_ref[...]) + bc2_ref[...]
    out = jnp.maximum(out, 0.0)
    out = _style(out, sty, l1W_ref[...], l1b_ref[...], a1Wg_ref[...],
                 a1Wb_ref[...], a1bg_ref[...], a1bb_ref[...], nzs1_ref[...])
    out = _style(out, sty, l2W_ref[...], l2b_ref[...], a2Wg_ref[...],
                 a2Wb_ref[...], a2bg_ref[...], a2bb_ref[...], nzs2_ref[...])
    out_ref[...] = out


_tc2_call = pl.pallas_call(
    _tc2_body,
    out_shape=jax.ShapeDtypeStruct((_N, _C), jnp.float32),
)


def kernel(x, edge_index, style, W_edge, b_edge, Wc1, bc1, Wc2, bc2,
           l0W, l0b, a0W, a0b, ns0, nz0,
           l1W, l1b, a1W, a1b, ns1, nz1,
           l2W, l2b, a2W, a2b, ns2, nz2):
    ei = edge_index.astype(jnp.int32)
    npad = _EPAD - _E
    srcp = jnp.concatenate([ei[0], jnp.zeros((npad,), jnp.int32)])
    dstp = jnp.concatenate([ei[1], jnp.zeros((npad,), jnp.int32)])
    zeros = jnp.zeros((_NP, _C), jnp.float32)
    acc = _sc_segment_sum(srcp, dstp, W_edge, zeros)

    def prep(aW, ab, ns, nz):
        return (aW[:_C], aW[_C:], ab[:_C].reshape(1, _C),
                ab[_C:].reshape(1, _C), (ns * nz).reshape(1, _C))

    a0Wg, a0Wb, a0bg, a0bb, nzs0 = prep(a0W, a0b, ns0, nz0)
    a1Wg, a1Wb, a1bg, a1bb, nzs1 = prep(a1W, a1b, ns1, nz1)
    a2Wg, a2Wb, a2bg, a2bb, nzs2 = prep(a2W, a2b, ns2, nz2)

    xm = _tc1_call(x, style, l0W, l0b.reshape(1, _C), a0Wg, a0Wb,
                   a0bg, a0bb, nzs0)

    return _tc2_call(
        acc, xm, style, b_edge.reshape(1, _C), Wc1, bc1.reshape(1, _C),
        Wc2, bc2.reshape(1, _C),
        l1W, l1b.reshape(1, _C), a1Wg, a1Wb, a1bg, a1bb, nzs1,
        l2W, l2b.reshape(1, _C), a2Wg, a2Wb, a2bg, a2bb, nzs2)
